# baseline (speedup 1.0000x reference)
import functools

import jax
import jax.numpy as jnp
from jax import lax
from jax.experimental import pallas as pl
from jax.experimental.pallas import tpu as pltpu
from jax.experimental.pallas import tpu_sc as plsc

VOCAB = 1000000
DIM = 64

NC = 2
NS = 16
NW = NC * NS

B_TOTAL = 4096 * 200
B_PER_W = B_TOTAL // NW
CHUNK = 128
NCHUNK = B_PER_W // CHUNK
NBUF = 4
NITER = NCHUNK // NBUF


def _make_kernel():
  mesh = plsc.VectorSubcoreMesh(core_axis_name="c", subcore_axis_name="s")

  @functools.partial(
      pl.kernel,
      mesh=mesh,
      compiler_params=pltpu.CompilerParams(use_tc_tiling_on_sc=False),
      out_type=jax.ShapeDtypeStruct((NW, NCHUNK, CHUNK, DIM), jnp.float32),
      scratch_types=[
          pltpu.VMEM((NCHUNK, CHUNK), jnp.int32),
          pltpu.VMEM((NBUF, CHUNK, DIM), jnp.float32),
          pltpu.SemaphoreType.DMA((NBUF,)),
          pltpu.SemaphoreType.DMA((NBUF,)),
      ],
  )
  def gather_kernel(idx_hbm, table_hbm, out_hbm, idx_v, bufs, gsem, ssem):
    wid = lax.axis_index("s") * NC + lax.axis_index("c")

    pltpu.sync_copy(idx_hbm.at[wid], idx_v)

    def issue_gather(b, c):
      pltpu.async_copy(table_hbm.at[idx_v.at[c]], bufs.at[b], gsem.at[b])

    def wait_gather(b):
      pltpu.make_async_copy(
          table_hbm.at[idx_v.at[0]], bufs.at[b], gsem.at[b]).wait()

    def issue_scatter(b, c):
      pltpu.async_copy(bufs.at[b], out_hbm.at[wid, c], ssem.at[b])

    def wait_scatter(b):
      pltpu.make_async_copy(
          bufs.at[b], out_hbm.at[wid, 0], ssem.at[b]).wait()

    for b in range(NBUF):
      issue_gather(b, b)

    def body(i, _):
      for b in range(NBUF):
        wait_gather(b)
        issue_scatter(b, (i - 1) * NBUF + b)
      for b in range(NBUF):
        wait_scatter(b)
        issue_gather(b, i * NBUF + b)
      return _

    lax.fori_loop(1, NITER, body, 0, unroll=False)

    for b in range(NBUF):
      wait_gather(b)
      issue_scatter(b, (NITER - 1) * NBUF + b)
    for b in range(NBUF):
      wait_scatter(b)

  return gather_kernel


_kernel = _make_kernel()


@jax.jit
def kernel(inputs, embed):
  idx = inputs.astype(jnp.int32).reshape(NW, NCHUNK, CHUNK)
  out = _kernel(idx, embed)
  return out.reshape(inputs.shape[0], inputs.shape[1], DIM)


# speedup vs baseline: 1.0031x; 1.0031x over previous
import functools

import jax
import jax.numpy as jnp
from jax import lax
from jax.experimental import pallas as pl
from jax.experimental.pallas import tpu as pltpu
from jax.experimental.pallas import tpu_sc as plsc

VOCAB = 1000000
DIM = 64

NC = 2
NS = 16
NW = NC * NS

B_TOTAL = 4096 * 200
B_PER_W = B_TOTAL // NW
CHUNK = 128
NCHUNK = B_PER_W // CHUNK
NBUF = 8
LOOK = NBUF // 2


def _make_kernel():
  mesh = plsc.VectorSubcoreMesh(core_axis_name="c", subcore_axis_name="s")

  @functools.partial(
      pl.kernel,
      mesh=mesh,
      compiler_params=pltpu.CompilerParams(use_tc_tiling_on_sc=False),
      out_type=jax.ShapeDtypeStruct((NW, NCHUNK, CHUNK, DIM), jnp.float32),
      scratch_types=[
          pltpu.VMEM((NCHUNK, CHUNK), jnp.int32),
          pltpu.VMEM((NBUF, CHUNK, DIM), jnp.float32),
          pltpu.SemaphoreType.DMA((NBUF,)),
          pltpu.SemaphoreType.DMA((NBUF,)),
      ],
  )
  def gather_kernel(idx_hbm, table_hbm, out_hbm, idx_v, bufs, gsem, ssem):
    wid = lax.axis_index("s") * NC + lax.axis_index("c")

    pltpu.sync_copy(idx_hbm.at[wid], idx_v)

    def issue_gather(b, c):
      pltpu.async_copy(table_hbm.at[idx_v.at[c]], bufs.at[b], gsem.at[b])

    def wait_gather(b):
      pltpu.make_async_copy(
          table_hbm.at[idx_v.at[0]], bufs.at[b], gsem.at[b]).wait()

    def issue_scatter(b, c):
      pltpu.async_copy(bufs.at[b], out_hbm.at[wid, c], ssem.at[b])

    def wait_scatter(b):
      pltpu.make_async_copy(
          bufs.at[b], out_hbm.at[wid, 0], ssem.at[b]).wait()


    def step(c, b):
      b2 = (b + LOOK) % NBUF
      wait_scatter(b2)
      issue_gather(b2, c + LOOK)
      wait_gather(b)
      issue_scatter(b, c)

    for c in range(LOOK):
      issue_gather(c % NBUF, c)
    for c in range(LOOK, NBUF):
      issue_gather(c % NBUF, c)
      wait_gather((c - LOOK) % NBUF)
      issue_scatter((c - LOOK) % NBUF, c - LOOK)

    def body(i, _):
      base = LOOK + (i - 1) * NBUF
      for j in range(NBUF):
        c = base + j
        step(c, (LOOK + j) % NBUF)
      return _

    n_steady = (NCHUNK - NBUF)
    assert n_steady % NBUF == 0
    lax.fori_loop(1, n_steady // NBUF + 1, body, 0, unroll=False)

    for c in range(NCHUNK - LOOK, NCHUNK):
      b = c % NBUF
      b2 = (b + LOOK) % NBUF
      wait_scatter(b2)
      wait_gather(b)
      issue_scatter(b, c)
    for c in range(NCHUNK - LOOK, NCHUNK):
      wait_scatter(c % NBUF)

  return gather_kernel


_kernel = _make_kernel()


@jax.jit
def kernel(inputs, embed):
  idx = inputs.astype(jnp.int32).reshape(NW, NCHUNK, CHUNK)
  out = _kernel(idx, embed)
  return out.reshape(inputs.shape[0], inputs.shape[1], DIM)
